# baseline (speedup 1.0000x reference)
import functools
import jax
import jax.numpy as jnp
from jax import lax
from jax.experimental import pallas as pl
from jax.experimental.pallas import tpu as pltpu, tpu_sc as plsc

B = 256
M = 16384
NW = 4
NR = 8

NWORK = 32
BPW = B // NWORK
CH = 2048
CPB = M // CH
NCH = BPW * CPB
UNROLL = 4
L = 16


def _sc_body(ww_hbm, fg_hbm, rw_hbm, prev_hbm, out_hbm,
             ww_v, rw_v, prev_v, out_v, fg_v,
             in_sem0, in_sem1, out_sem0, out_sem1):
    in_sems = (in_sem0, in_sem1)
    out_sems = (out_sem0, out_sem1)
    wid = lax.axis_index("s") * 2 + lax.axis_index("c")
    b0 = wid * BPW

    pltpu.sync_copy(fg_hbm.at[pl.ds(b0 * NR, BPW * NR)], fg_v.at[pl.ds(0, BPW * NR)])

    def start_in(c, s):
        b = b0 + c // CPB
        m0 = (c % CPB) * CH
        pltpu.async_copy(ww_hbm.at[b, :, pl.ds(m0, CH)], ww_v.at[s], in_sems[s])
        pltpu.async_copy(rw_hbm.at[b, :, pl.ds(m0, CH)], rw_v.at[s], in_sems[s])
        pltpu.async_copy(prev_hbm.at[b, pl.ds(m0, CH)], prev_v.at[s], in_sems[s])

    def wait_in(s):
        pltpu.make_async_copy(ww_hbm.at[0, :, pl.ds(0, CH)], ww_v.at[s], in_sems[s]).wait()
        pltpu.make_async_copy(rw_hbm.at[0, :, pl.ds(0, CH)], rw_v.at[s], in_sems[s]).wait()
        pltpu.make_async_copy(prev_hbm.at[0, pl.ds(0, CH)], prev_v.at[s], in_sems[s]).wait()

    def start_out(c, s):
        b = b0 + c // CPB
        m0 = (c % CPB) * CH
        pltpu.async_copy(out_v.at[s], out_hbm.at[b, pl.ds(m0, CH)], out_sems[s])

    def wait_out(s):
        pltpu.make_async_copy(out_v.at[s], out_hbm.at[0, pl.ds(0, CH)], out_sems[s]).wait()

    start_in(0, 0)
    start_in(1, 1)

    def group_body(g, carry):
        for s in range(2):
            c = 2 * g + s
            wait_in(s)
            bi = c // CPB
            fvec = fg_v[pl.ds(bi * NR, L)]
            fgs = [fvec[r] for r in range(NR)]

            @pl.when(g >= 1)
            def _():
                wait_out(s)

            def vec_body(i, carry2):
                for u in range(UNROLL):
                    sl = pl.ds((i * UNROLL + u) * L, L)
                    p = (1.0 - ww_v[s, 0, sl]) * (1.0 - ww_v[s, 1, sl]) \
                        * (1.0 - ww_v[s, 2, sl]) * (1.0 - ww_v[s, 3, sl])
                    acc = (1.0 - prev_v[s, sl]) * p
                    free = fgs[0] * rw_v[s, 0, sl]
                    for r in range(1, NR):
                        free = free + fgs[r] * rw_v[s, r, sl]
                    res = 1.0 - acc - free
                    out_v[s, sl] = jnp.minimum(jnp.maximum(res, 0.0), 1.0)
                return carry2

            lax.fori_loop(0, CH // (L * UNROLL), vec_body, 0, unroll=False)

            start_out(c, s)

            @pl.when(g < NCH // 2 - 1)
            def _():
                start_in(c + 2, s)
        return carry

    lax.fori_loop(0, NCH // 2, group_body, 0, unroll=False)
    wait_out(0)
    wait_out(1)


def kernel(write_weights, free_gate, read_weights, prev_usage):
    mesh = plsc.VectorSubcoreMesh(core_axis_name="c", subcore_axis_name="s")
    f32 = jnp.float32
    k = functools.partial(
        pl.kernel,
        mesh=mesh,
        out_type=jax.ShapeDtypeStruct((B, M), f32),
        scratch_types=[
            pltpu.VMEM((2, NW, CH), f32),
            pltpu.VMEM((2, NR, CH), f32),
            pltpu.VMEM((2, CH), f32),
            pltpu.VMEM((2, CH), f32),
            pltpu.VMEM((BPW * NR + L, ), f32),
            pltpu.SemaphoreType.DMA,
            pltpu.SemaphoreType.DMA,
            pltpu.SemaphoreType.DMA,
            pltpu.SemaphoreType.DMA,
        ],
    )(_sc_body)
    return k(write_weights, free_gate.reshape(B * NR), read_weights, prev_usage)
